# baseline (speedup 1.0000x reference)
import functools

import jax
import jax.numpy as jnp
from jax import lax
from jax.experimental import pallas as pl
from jax.experimental.pallas import tpu as pltpu
from jax.experimental.pallas import tpu_sc as plsc

_S = 50
_BX = 16384
_D = 64
_V2 = 500000
_BLK = 128
_NC, _NS = 2, 16
_NW = _NC * _NS
_TASKS = _S * (_BX // _BLK)
_TPW = _TASKS // _NW

_mesh = plsc.VectorSubcoreMesh(core_axis_name="c", subcore_axis_name="s")


@functools.partial(
    pl.kernel,
    mesh=_mesh,
    out_type=jax.ShapeDtypeStruct((_S, _D, _BX), jnp.float32),
    scratch_types=[
        pltpu.VMEM((2, _BLK), jnp.int32),
        pltpu.VMEM((2, _BLK), jnp.int32),
        pltpu.VMEM((2, _BLK), jnp.int32),
        pltpu.VMEM((2, _BLK, 128), jnp.float32),
        pltpu.VMEM((2, _D, _BLK), jnp.float32),
        pltpu.SemaphoreType.DMA((2,)),
        pltpu.SemaphoreType.DMA((2,)),
        pltpu.SemaphoreType.DMA((2,)),
    ],
    compiler_params=pltpu.CompilerParams(use_tc_tiling_on_sc=True,
                                         needs_layout_passes=False),
)
def _embed(xt_hbm, t2_hbm, out_hbm, idx_v, sidx_v, pb_v, gbuf, tbuf,
           sem_i, sem_g, sem_w):
    wid = lax.axis_index("s") * _NC + lax.axis_index("c")
    t0 = wid * _TPW

    def task_sb(t):
        s = t // (_BX // _BLK)
        b0 = (t % (_BX // _BLK)) * _BLK
        return s, b0

    def idx_copy(t, b):
        s, b0 = task_sb(t)
        return pltpu.make_async_copy(
            xt_hbm.at[s, pl.ds(b0, _BLK)], idx_v.at[b], sem_i.at[b])

    def gather_copy(b):
        return pltpu.make_async_copy(
            t2_hbm.at[sidx_v.at[b]], gbuf.at[b], sem_g.at[b])

    def write_copy(t, b):
        s, b0 = task_sb(t)
        return pltpu.make_async_copy(
            tbuf.at[b], out_hbm.at[s, :, pl.ds(b0, _BLK)], sem_w.at[b])

    def prep_indices(b):
        for g in range(8):
            sl = pl.ds(16 * g, 16)
            raw = idx_v[b, sl]
            sidx_v[b, sl] = lax.shift_right_logical(raw, 1)
            pb_v[b, sl] = lax.shift_left(raw & 1, 6)

    idx_copy(t0, 0).start()
    idx_copy(t0 + 1, 1).start()
    idx_copy(t0, 0).wait()
    prep_indices(0)
    gather_copy(0).start()

    def body(i, carry):
        t = t0 + i
        b = lax.rem(i, 2)

        @pl.when(i + 1 < _TPW)
        def _():
            bn = lax.rem(i + 1, 2)
            idx_copy(t + 1, bn).wait()
            prep_indices(bn)

            @pl.when(i + 2 < _TPW)
            def _():
                idx_copy(t + 2, b).start()

            gather_copy(bn).start()

        gather_copy(b).wait()

        @pl.when(i >= 2)
        def _():
            write_copy(t - 2, b).wait()

        iota = lax.broadcasted_iota(jnp.int32, (16,), 0)
        for g in range(8):
            jbase = iota + (16 * g)
            pb = pb_v[b, pl.ds(16 * g, 16)]

            def dbody(d, _, jbase=jbase, pb=pb, b=b):
                diag = (iota + d) & 15
                for cg in range(4):
                    c_vec = diag + (16 * cg)
                    vals = plsc.load_gather(gbuf.at[b], [jbase, pb + c_vec])
                    plsc.store_scatter(tbuf.at[b], [c_vec, jbase], vals)
                return _

            lax.fori_loop(0, 16, dbody, 0)

        write_copy(t, b).start()
        return carry

    lax.fori_loop(0, _TPW, body, 0)

    write_copy(t0 + _TPW - 2, lax.rem(_TPW - 2, 2)).wait()
    write_copy(t0 + _TPW - 1, lax.rem(_TPW - 1, 2)).wait()


def kernel(x, embedding_table):
    xt = x.T
    t2 = embedding_table.reshape(_V2, 128)
    ot = _embed(xt, t2)
    return jnp.transpose(ot, (2, 0, 1))


# speedup vs baseline: 1.0137x; 1.0137x over previous
import functools

import jax
import jax.numpy as jnp
from jax import lax
from jax.experimental import pallas as pl
from jax.experimental.pallas import tpu as pltpu
from jax.experimental.pallas import tpu_sc as plsc

_S = 50
_BX = 16384
_D = 64
_V2 = 500000
_BLK = 128
_NC, _NS = 2, 16
_NW = _NC * _NS
_TASKS = _S * (_BX // _BLK)
_TPW = _TASKS // _NW

_mesh = plsc.VectorSubcoreMesh(core_axis_name="c", subcore_axis_name="s")


@functools.partial(
    pl.kernel,
    mesh=_mesh,
    out_type=jax.ShapeDtypeStruct((_S, _D, _BX), jnp.float32),
    scratch_types=[
        pltpu.VMEM((2, _BLK), jnp.int32),
        pltpu.VMEM((2, _BLK), jnp.int32),
        pltpu.VMEM((2, _BLK), jnp.int32),
        pltpu.VMEM((2, _BLK, 128), jnp.float32),
        pltpu.VMEM((2, _D, _BLK), jnp.float32),
        pltpu.SemaphoreType.DMA((2,)),
        pltpu.SemaphoreType.DMA((2,)),
        pltpu.SemaphoreType.DMA((2,)),
    ],
    compiler_params=pltpu.CompilerParams(use_tc_tiling_on_sc=True,
                                         needs_layout_passes=False),
)
def _embed(xt_hbm, t2_hbm, out_hbm, idx_v, sidx_v, pb_v, gbuf, tbuf,
           sem_i, sem_g, sem_w):
    wid = lax.axis_index("s") * _NC + lax.axis_index("c")
    t0 = wid * _TPW

    def task_sb(t):
        s = t // (_BX // _BLK)
        b0 = (t % (_BX // _BLK)) * _BLK
        return s, b0

    def idx_copy(t, b):
        s, b0 = task_sb(t)
        return pltpu.make_async_copy(
            xt_hbm.at[s, pl.ds(b0, _BLK)], idx_v.at[b], sem_i.at[b])

    def gather_copy(b):
        return pltpu.make_async_copy(
            t2_hbm.at[sidx_v.at[b]], gbuf.at[b], sem_g.at[b])

    def write_copy(t, b):
        s, b0 = task_sb(t)
        return pltpu.make_async_copy(
            tbuf.at[b], out_hbm.at[s, :, pl.ds(b0, _BLK)], sem_w.at[b])

    def prep_indices(b):
        for g in range(8):
            sl = pl.ds(16 * g, 16)
            raw = idx_v[b, sl]
            sidx_v[b, sl] = lax.shift_right_logical(raw, 1)
            pb_v[b, sl] = lax.shift_left(raw & 1, 6)

    idx_copy(t0, 0).start()
    idx_copy(t0 + 1, 1).start()
    idx_copy(t0, 0).wait()
    prep_indices(0)
    gather_copy(0).start()

    def body(i, carry):
        t = t0 + i
        b = lax.rem(i, 2)

        @pl.when(i + 1 < _TPW)
        def _():
            bn = lax.rem(i + 1, 2)
            idx_copy(t + 1, bn).wait()
            prep_indices(bn)

            @pl.when(i + 2 < _TPW)
            def _():
                idx_copy(t + 2, b).start()

            gather_copy(bn).start()

        gather_copy(b).wait()

        @pl.when(i >= 2)
        def _():
            write_copy(t - 2, b).wait()

        iota = lax.broadcasted_iota(jnp.int32, (16,), 0)

        def gbody(g, _, b=b):
            jbase = iota + g * 16
            pb = pb_v[b, pl.ds(g * 16, 16)]
            for d in range(16):
                diag = (iota + d) & 15
                for cg in range(4):
                    c_vec = diag + (16 * cg)
                    vals = plsc.load_gather(gbuf.at[b], [jbase, pb + c_vec])
                    plsc.store_scatter(tbuf.at[b], [c_vec, jbase], vals)
            return _

        lax.fori_loop(0, 8, gbody, 0)

        write_copy(t, b).start()
        return carry

    lax.fori_loop(0, _TPW, body, 0)

    write_copy(t0 + _TPW - 2, lax.rem(_TPW - 2, 2)).wait()
    write_copy(t0 + _TPW - 1, lax.rem(_TPW - 1, 2)).wait()


def kernel(x, embedding_table):
    xt = x.T
    t2 = embedding_table.reshape(_V2, 128)
    ot = _embed(xt, t2)
    return jnp.transpose(ot, (2, 0, 1))


# speedup vs baseline: 1.0853x; 1.0707x over previous
import functools

import jax
import jax.numpy as jnp
from jax import lax
from jax.experimental import pallas as pl
from jax.experimental.pallas import tpu as pltpu
from jax.experimental.pallas import tpu_sc as plsc

_S = 50
_BX = 16384
_D = 64
_V2 = 500000
_BLK = 128
_NC, _NS = 2, 16
_NW = _NC * _NS
_TASKS = _S * (_BX // _BLK)
_TPW = _TASKS // _NW

_mesh = plsc.VectorSubcoreMesh(core_axis_name="c", subcore_axis_name="s")


@functools.partial(
    pl.kernel,
    mesh=_mesh,
    out_type=jax.ShapeDtypeStruct((_S, _D, _BX), jnp.float32),
    scratch_types=[
        pltpu.VMEM((2, _BLK), jnp.int32),
        pltpu.VMEM((2, _BLK), jnp.int32),
        pltpu.VMEM((2, _BLK), jnp.int32),
        pltpu.VMEM((2, _BLK, 128), jnp.float32),
        pltpu.VMEM((2, _D, _BLK), jnp.float32),
        pltpu.SemaphoreType.DMA((2,)),
        pltpu.SemaphoreType.DMA((2,)),
        pltpu.SemaphoreType.DMA((2,)),
    ],
    compiler_params=pltpu.CompilerParams(use_tc_tiling_on_sc=True,
                                         needs_layout_passes=False),
)
def _embed(xt_hbm, t2_hbm, out_hbm, idx_v, sidx_v, pb_v, gbuf, tbuf,
           sem_i, sem_g, sem_w):
    wid = lax.axis_index("s") * _NC + lax.axis_index("c")
    t0 = wid * _TPW

    def task_sb(t):
        s = t // (_BX // _BLK)
        b0 = (t % (_BX // _BLK)) * _BLK
        return s, b0

    def idx_copy(t, b):
        s, b0 = task_sb(t)
        return pltpu.make_async_copy(
            xt_hbm.at[s, pl.ds(b0, _BLK)], idx_v.at[b], sem_i.at[b])

    def gather_copy(b):
        return pltpu.make_async_copy(
            t2_hbm.at[sidx_v.at[b]], gbuf.at[b], sem_g.at[b])

    def write_copy(t, b):
        s, b0 = task_sb(t)
        return pltpu.make_async_copy(
            tbuf.at[b], out_hbm.at[s, :, pl.ds(b0, _BLK)], sem_w.at[b])

    def prep_indices(b):
        for g in range(8):
            sl = pl.ds(16 * g, 16)
            raw = idx_v[b, sl]
            sidx_v[b, sl] = lax.shift_right_logical(raw, 1)
            pb_v[b, sl] = lax.shift_left(raw & 1, 6)

    idx_copy(t0, 0).start()
    idx_copy(t0 + 1, 1).start()
    idx_copy(t0, 0).wait()
    prep_indices(0)
    gather_copy(0).start()

    def body(i, carry):
        t = t0 + i
        b = lax.rem(i, 2)

        @pl.when(i + 1 < _TPW)
        def _():
            bn = lax.rem(i + 1, 2)
            idx_copy(t + 1, bn).wait()
            prep_indices(bn)

            @pl.when(i + 2 < _TPW)
            def _():
                idx_copy(t + 2, b).start()

            gather_copy(bn).start()

        gather_copy(b).wait()

        @pl.when(i >= 2)
        def _():
            write_copy(t - 2, b).wait()

        iota = lax.broadcasted_iota(jnp.int32, (16,), 0)

        def gbody(g, _, b=b):
            jbase = iota + g * 16
            pb = pb_v[b, pl.ds(g * 16, 16)]
            for d in range(16):
                diag = (iota + d) & 15
                for cg in range(4):
                    c_vec = diag + (16 * cg)
                    vals = plsc.load_gather(gbuf.at[b], [jbase, pb + c_vec])
                    plsc.store_scatter(tbuf.at[b], [c_vec, jbase], vals)
            return _

        lax.fori_loop(0, 8, gbody, 0)

        write_copy(t, b).start()
        return carry

    lax.fori_loop(0, _TPW, body, 0)

    write_copy(t0 + _TPW - 2, lax.rem(_TPW - 2, 2)).wait()
    write_copy(t0 + _TPW - 1, lax.rem(_TPW - 1, 2)).wait()


_NBLK = 7811
_TAILR = _NBLK * 128


@functools.partial(
    pl.kernel,
    mesh=_mesh,
    out_type=jax.ShapeDtypeStruct((_V2, 128), jnp.float32),
    scratch_types=[
        pltpu.VMEM((2, _D, 192), jnp.float32),
        pltpu.VMEM((2, 96, 128), jnp.float32),
        pltpu.SemaphoreType.DMA((2,)),
        pltpu.SemaphoreType.DMA((2,)),
    ],
    compiler_params=pltpu.CompilerParams(use_tc_tiling_on_sc=True,
                                         needs_layout_passes=False),
)
def _format(tt_hbm, t2_hbm, blk, sbuf, sem_r, sem_w):
    wid = lax.axis_index("s") * _NC + lax.axis_index("c")
    iota = lax.broadcasted_iota(jnp.int32, (16,), 0)

    def read_copy(b, p):
        return pltpu.make_async_copy(
            tt_hbm.at[:, pl.ds(b * 128, 128)],
            blk.at[p, :, pl.ds(0, 128)], sem_r.at[p])

    def write_copy(b, p):
        return pltpu.make_async_copy(
            sbuf.at[p, pl.ds(0, _D), :],
            t2_hbm.at[pl.ds(b * _D, _D)], sem_w.at[p])

    def transpose(p, njg):
        def gbody(jg, _):
            for d in range(16):
                j_vec = jg * 16 + ((iota + d) & 15)
                j64 = lax.shift_left(j_vec, 6)
                for cg in range(4):
                    c_vec = iota + (16 * cg)
                    vals = plsc.load_gather(blk.at[p], [c_vec, j_vec])
                    flat = j64 + c_vec
                    plsc.store_scatter(
                        sbuf.at[p],
                        [lax.shift_right_logical(flat, 7), flat & 127], vals)
            return _

        lax.fori_loop(0, njg, gbody, 0)

    read_copy(wid, 0).start()

    def body(i, carry):
        b = wid + 32 * i
        p = lax.rem(i, 2)

        @pl.when(b < _NBLK)
        def _():
            read_copy(b, p).wait()

            @pl.when(b + 32 < _NBLK)
            def _():
                read_copy(b + 32, 1 - p).start()

            @pl.when(i >= 2)
            def _():
                write_copy(b - 64, p).wait()

            transpose(p, 8)
            write_copy(b, p).start()

        return carry

    lax.fori_loop(0, 245, body, 0)

    for io in (243, 244):
        bio = wid + 32 * io

        @pl.when(bio < _NBLK)
        def _(bio=bio, io=io):
            write_copy(bio, lax.rem(io, 2)).wait()

    @pl.when(jnp.logical_and(wid + 32 * 244 >= _NBLK, wid + 32 * 242 < _NBLK))
    def _():
        write_copy(wid + 32 * 242, 0).wait()

    @pl.when(wid == 31)
    def _():
        pltpu.sync_copy(tt_hbm.at[:, pl.ds(_TAILR, 192)], blk.at[0])
        transpose(0, 12)
        pltpu.sync_copy(sbuf.at[0], t2_hbm.at[pl.ds(_TAILR // 2, 96)])


def kernel(x, embedding_table):
    xt = x.T
    tt = embedding_table.T
    t2 = _format(tt)
    ot = _embed(xt, t2)
    return jnp.transpose(ot, (2, 0, 1))
